# speedup vs baseline: 5.7993x; 5.7993x over previous
import functools

import jax
import jax.numpy as jnp
from jax import lax
from jax.experimental import pallas as pl
from jax.experimental.pallas import tpu as pltpu
from jax.experimental.pallas import tpu_sc as plsc

NC = 2
NS = 16
NW = NC * NS
CH = 128


@functools.partial(jax.jit, static_argnames=("n_rows", "dim"))
def _sc_gather(idx, table, n_rows, dim):
    b_per_w = n_rows // NW
    n_chunks = b_per_w // CH

    def body(table_hbm, idx_hbm, out_hbm, idx_v, rows_v, sem):
        wid = lax.axis_index("s") * NC + lax.axis_index("c")
        base = wid * b_per_w

        @pl.loop(0, n_chunks)
        def _(i):
            off = base + i * CH
            pltpu.sync_copy(idx_hbm.at[pl.ds(off, CH)], idx_v)
            pltpu.async_copy(table_hbm.at[idx_v], rows_v, sem).wait()
            pltpu.sync_copy(rows_v, out_hbm.at[pl.ds(off, CH)])

    mesh = plsc.VectorSubcoreMesh(core_axis_name="c", subcore_axis_name="s")
    f = pl.kernel(
        body,
        out_type=jax.ShapeDtypeStruct((n_rows, dim), jnp.float32),
        mesh=mesh,
        scratch_types=[
            pltpu.VMEM((CH,), jnp.int32),
            pltpu.VMEM((CH, dim), jnp.float32),
            pltpu.SemaphoreType.DMA,
        ],
    )
    return f(table, idx)


def kernel(x, embedding):
    b, h = x.shape
    v, d = embedding.shape
    n_rows = b * h
    assert n_rows % (NW * CH) == 0
    xf = x.reshape(n_rows)
    out = _sc_gather(xf, embedding, n_rows, d)
    return out.reshape(b, h, d)


# speedup vs baseline: 16.8586x; 2.9070x over previous
import functools

import jax
import jax.numpy as jnp
from jax import lax
from jax.experimental import pallas as pl
from jax.experimental.pallas import tpu as pltpu
from jax.experimental.pallas import tpu_sc as plsc

NC = 2
NS = 16
NW = NC * NS
CH = 128


@functools.partial(jax.jit, static_argnames=("n_rows", "dim", "vocab"))
def _sc_gather(idx2d, table, n_rows, dim, vocab):
    b_per_w = n_rows // NW
    n_chunks = b_per_w // CH
    v_main = (vocab // (8 * NS)) * 8
    v_rem = vocab - v_main * NS

    def body(table_hbm, idx_hbm, out_hbm, shared_tab,
             ib0, ib1, rb0, rb1, gsem0, gsem1, isem0, isem1):
        cid = lax.axis_index("c")
        sid = lax.axis_index("s")
        wid = sid * NC + cid
        base = wid * b_per_w

        pltpu.sync_copy(
            table_hbm.at[pl.ds(sid * v_main, v_main)],
            shared_tab.at[pl.ds(sid * v_main, v_main)],
        )
        if v_rem:
            @pl.when(sid == 0)
            def _():
                pltpu.sync_copy(
                    table_hbm.at[pl.ds(NS * v_main, v_rem)],
                    shared_tab.at[pl.ds(NS * v_main, v_rem)],
                )
        plsc.subcore_barrier()

        ibs = (ib0, ib1)
        rbs = (rb0, rb1)
        gsems = (gsem0, gsem1)
        isems = (isem0, isem1)
        chunk0 = wid * n_chunks

        for b in range(2):
            pltpu.sync_copy(idx_hbm.at[pl.ds(chunk0 + b, 1)], ibs[b])
            pltpu.async_copy(shared_tab.at[ibs[b].at[0]], rbs[b], gsems[b])

        @pl.loop(0, n_chunks, step=2)
        def _(j):
            for b in range(2):
                jj = j + b
                ib, rb, gsem, isem = ibs[b], rbs[b], gsems[b], isems[b]
                pltpu.make_async_copy(shared_tab.at[ib.at[0]], rb, gsem).wait()

                @pl.when(jj + 2 < n_chunks)
                def _():
                    pltpu.async_copy(idx_hbm.at[pl.ds(chunk0 + jj + 2, 1)], ib, isem)

                pltpu.sync_copy(rb, out_hbm.at[pl.ds(base + jj * CH, CH)])

                @pl.when(jj + 2 < n_chunks)
                def _():
                    pltpu.make_async_copy(
                        idx_hbm.at[pl.ds(chunk0 + jj + 2, 1)], ib, isem).wait()
                    pltpu.async_copy(shared_tab.at[ib.at[0]], rb, gsem)

    mesh = plsc.VectorSubcoreMesh(core_axis_name="c", subcore_axis_name="s")
    f = pl.kernel(
        body,
        out_type=jax.ShapeDtypeStruct((n_rows, dim), jnp.float32),
        mesh=mesh,
        scratch_types=[
            pltpu.VMEM_SHARED((vocab, dim), jnp.float32),
            pltpu.VMEM((1, CH), jnp.int32),
            pltpu.VMEM((1, CH), jnp.int32),
            pltpu.VMEM((CH, dim), jnp.float32),
            pltpu.VMEM((CH, dim), jnp.float32),
            pltpu.SemaphoreType.DMA,
            pltpu.SemaphoreType.DMA,
            pltpu.SemaphoreType.DMA,
            pltpu.SemaphoreType.DMA,
        ],
    )
    return f(table, idx2d)


def kernel(x, embedding):
    b, h = x.shape
    v, d = embedding.shape
    n_rows = b * h
    assert n_rows % (NW * CH * 2) == 0
    idx2d = x.reshape(n_rows // CH, CH)
    out = _sc_gather(idx2d, embedding, n_rows, d, v)
    return out.reshape(b, h, d)
